# Initial kernel scaffold; baseline (speedup 1.0000x reference)
import jax
import jax.numpy as jnp
from jax.experimental import pallas as pl


def kernel(x, edge_index, W1l, b1, W1r, W2l, b2, W2r):
    raise NotImplementedError("write your pallas kernel here")


# speedup vs baseline: 5.0226x; 5.0226x over previous
import functools

import jax
import jax.numpy as jnp
from jax import lax
from jax.experimental import pallas as pl
from jax.experimental.pallas import tpu as pltpu
from jax.experimental.pallas import tpu_sc as plsc

N_NODES = 10000
N_EDGES = 320000
D = 128

NC = 2
NS = 16
NW = NC * NS

CH = 128
KPW = -(-N_EDGES // (NW * CH))
EPW = KPW * CH
EPAD = NW * EPW

NPAD = 10240
STRIPE = NPAD // NS

_f32 = jnp.float32


def _sc_body(want_cnt, *refs):
    if want_cnt:
        (table, srcs, dsts, zrows, zcnt, aggp, cntp,
         src_v, dst_v, rows, ones_v, sh_agg, sh_cnt, sem) = refs
    else:
        (table, srcs, dsts, zrows, aggp,
         src_v, dst_v, rows, sh_agg, sem) = refs
    c = lax.axis_index("c")
    s = lax.axis_index("s")
    w = c * NS + s
    r0 = s * STRIPE

    pltpu.sync_copy(zrows.at[pl.ds(r0, STRIPE)], sh_agg.at[pl.ds(r0, STRIPE)])
    if want_cnt:
        pltpu.sync_copy(zcnt.at[pl.ds(r0, STRIPE)], sh_cnt.at[pl.ds(r0, STRIPE)])
        for i in range(CH // 16):
            ones_v[pl.ds(i * 16, 16)] = jnp.ones((16,), _f32)

    pltpu.sync_copy(srcs.at[w], src_v)
    pltpu.sync_copy(dsts.at[w], dst_v)
    plsc.subcore_barrier()

    def chunk(j, carry):
        pltpu.async_copy(table.at[src_v.at[j]], rows, sem).wait()
        pltpu.sync_copy(rows, sh_agg.at[dst_v.at[j]], add=True)
        if want_cnt:
            pltpu.sync_copy(ones_v, sh_cnt.at[dst_v.at[j]], add=True)
        return carry

    lax.fori_loop(0, KPW, chunk, 0)
    plsc.subcore_barrier()

    pltpu.sync_copy(sh_agg.at[pl.ds(r0, STRIPE)], aggp.at[c].at[pl.ds(r0, STRIPE)])
    if want_cnt:
        pltpu.sync_copy(sh_cnt.at[pl.ds(r0, STRIPE)], cntp.at[c].at[pl.ds(r0, STRIPE)])


def _make_sc_kernel(want_cnt):
    out_type = [jax.ShapeDtypeStruct((NC, NPAD, D), _f32)]
    if want_cnt:
        out_type.append(jax.ShapeDtypeStruct((NC, NPAD), _f32))
    scratch = [
        pltpu.VMEM((KPW, CH), jnp.int32),
        pltpu.VMEM((KPW, CH), jnp.int32),
        pltpu.VMEM((CH, D), _f32),
    ]
    if want_cnt:
        scratch.append(pltpu.VMEM((CH,), _f32))
    scratch.append(pltpu.VMEM_SHARED((NPAD, D), _f32))
    if want_cnt:
        scratch.append(pltpu.VMEM_SHARED((NPAD,), _f32))
    scratch.append(pltpu.SemaphoreType.DMA)
    mesh = plsc.VectorSubcoreMesh(core_axis_name="c", subcore_axis_name="s")
    return pl.kernel(
        functools.partial(_sc_body, want_cnt),
        out_type=tuple(out_type) if want_cnt else out_type[0],
        mesh=mesh,
        scratch_types=scratch,
        name="sage_sc_agg" + ("_cnt" if want_cnt else ""),
    )


_sc_agg_cnt = _make_sc_kernel(True)
_sc_agg = _make_sc_kernel(False)


def _tc_body(relu, aggp, cnt3, x, wl, bl, wr, out):
    agg = aggp[0] + aggp[1]
    cnt = cnt3[0] + cnt3[1]
    mean = agg / jnp.maximum(cnt, 1.0)
    h = (lax.dot_general(mean, wl[...], (((1,), (1,)), ((), ())),
                         preferred_element_type=_f32)
         + bl[...]
         + lax.dot_general(x[...], wr[...], (((1,), (1,)), ((), ())),
                           preferred_element_type=_f32))
    if relu:
        h = jnp.maximum(h, 0.0)
    out[...] = h


_TCR = 1000


def _tc_layer(aggp, cnt3, x, wl, bl, wr, relu):
    grid = N_NODES // _TCR
    return pl.pallas_call(
        functools.partial(_tc_body, relu),
        grid=(grid,),
        in_specs=[
            pl.BlockSpec((NC, _TCR, D), lambda i: (0, i, 0)),
            pl.BlockSpec((NC, _TCR, 1), lambda i: (0, i, 0)),
            pl.BlockSpec((_TCR, D), lambda i: (i, 0)),
            pl.BlockSpec((D, D), lambda i: (0, 0)),
            pl.BlockSpec((1, D), lambda i: (0, 0)),
            pl.BlockSpec((D, D), lambda i: (0, 0)),
        ],
        out_specs=pl.BlockSpec((_TCR, D), lambda i: (i, 0)),
        out_shape=jax.ShapeDtypeStruct((N_NODES, D), _f32),
        name="sage_tc_dense" + ("_relu" if relu else ""),
    )(aggp, cnt3, x, wl, bl, wr)


def kernel(x, edge_index, W1l, b1, W1r, W2l, b2, W2r):
    src = edge_index[0].astype(jnp.int32)
    dst = edge_index[1].astype(jnp.int32)
    pad = EPAD - N_EDGES
    src_p = jnp.concatenate([src, jnp.zeros((pad,), jnp.int32)]).reshape(NW, KPW, CH)
    dst_p = jnp.concatenate([dst, jnp.full((pad,), N_NODES, jnp.int32)]).reshape(NW, KPW, CH)
    zrows = jnp.zeros((NPAD, D), _f32)
    zcnt = jnp.zeros((NPAD,), _f32)

    aggp, cntp = _sc_agg_cnt(x, src_p, dst_p, zrows, zcnt)
    cnt3 = cntp.reshape(NC, NPAD, 1)
    b1r = b1.reshape(1, D)
    b2r = b2.reshape(1, D)

    h = _tc_layer(aggp, cnt3, x, W1l, b1r, W1r, relu=True)
    aggp2 = _sc_agg(h, src_p, dst_p, zrows)
    out = _tc_layer(aggp2, cnt3, h, W2l, b2r, W2r, relu=False)
    return out
